# baseline (speedup 1.0000x reference)
import functools

import jax
import jax.numpy as jnp
from jax import lax
from jax.experimental import pallas as pl
from jax.experimental.pallas import tpu as pltpu
from jax.experimental.pallas import tpu_sc as plsc

NC = 2
NS = 16
L = 16
NW = NC * NS

R = 32768
D = 1024
HALF = R // 2
SPAN = HALF // NW
CH = 32
NCH = SPAN // CH

_mesh = plsc.VectorSubcoreMesh(core_axis_name="c", subcore_axis_name="s")


@functools.partial(
    pl.kernel,
    mesh=_mesh,
    out_type=(
        jax.ShapeDtypeStruct((HALF, D), jnp.float32),
        jax.ShapeDtypeStruct((HALF, D), jnp.float32),
    ),
    scratch_types=[
        pltpu.VMEM((2 * SPAN,), jnp.float32),
        pltpu.VMEM((L,), jnp.float32),
        pltpu.VMEM((2 * NCH, CH), jnp.int32),
        pltpu.VMEM((CH, D), jnp.float32),
        pltpu.VMEM((CH, D), jnp.float32),
        pltpu.VMEM((CH, D), jnp.float32),
        pltpu.SemaphoreType.DMA,
        pltpu.SemaphoreType.DMA,
        pltpu.SemaphoreType.DMA,
        pltpu.SemaphoreType.DMA,
        pltpu.SemaphoreType.DMA,
        pltpu.SemaphoreType.DMA,
    ],
)
def _exchange_sc(x_hbm, mask_hbm, thr_hbm, out0_hbm, out1_hbm, mask_v, thr_v,
                 idx_v, buf0, buf1, buf2, gsem0, gsem1, gsem2,
                 ssem0, ssem1, ssem2):
    wid = lax.axis_index("s") * NC + lax.axis_index("c")
    base = wid * SPAN

    pltpu.sync_copy(mask_hbm.at[pl.ds(base, SPAN)], mask_v.at[pl.ds(0, SPAN)])
    pltpu.sync_copy(mask_hbm.at[pl.ds(HALF + base, SPAN)],
                    mask_v.at[pl.ds(SPAN, SPAN)])
    pltpu.sync_copy(thr_hbm, thr_v)
    thr = thr_v[...]

    iota = lax.iota(jnp.int32, L)
    for p in range(2):
        off_t = HALF if p else 0
        off_f = 0 if p else HALF
        for j in range(SPAN // L):
            m = mask_v[pl.ds(p * SPAN + j * L, L)]
            src = base + j * L + iota + jnp.where(m >= thr, off_t, off_f)
            idx_v[p * NCH + j // (CH // L), pl.ds((j % (CH // L)) * L, L)] = src

    bufs = (buf0, buf1, buf2)
    gsems = (gsem0, gsem1, gsem2)
    ssems = (ssem0, ssem1, ssem2)
    NBUF = 3
    G = (NCH // NBUF) - 1

    for p, out_hbm in enumerate((out0_hbm, out1_hbm)):

        def out_at(c):
            return out_hbm.at[pl.ds(pl.multiple_of(base + c * CH, 8), CH)]

        def idx_at(c):
            return idx_v.at[p * NCH + c]

        def start_gather(c, b):
            pltpu.async_copy(x_hbm.at[idx_at(c)], bufs[b], gsems[b])

        def wait_gather(c, b):
            pltpu.make_async_copy(x_hbm.at[idx_at(c)], bufs[b],
                                  gsems[b]).wait()

        def start_store(c, b):
            pltpu.async_copy(bufs[b], out_at(c), ssems[b])

        def wait_store(c, b):
            pltpu.make_async_copy(bufs[b], out_at(c), ssems[b]).wait()

        for b in range(NBUF):
            start_gather(b, b)

        def body(g, carry):
            for b in range(NBUF):
                c = g * NBUF + b
                wait_gather(c, b)
                start_store(c, b)
                wait_store(c, b)
                start_gather(c + NBUF, b)
            return carry

        lax.fori_loop(0, G, body, 0)

        for c in range(G * NBUF, NCH):
            b = c % NBUF
            wait_gather(c, b)
            start_store(c, b)
            if c + NBUF < NCH:
                wait_store(c, b)
                start_gather(c + NBUF, b)
        for c in range(NCH - NBUF, NCH):
            wait_store(c, c % NBUF)


def kernel(x, mask, mask_threshold):
    xf = x.reshape(R, D)
    mf = mask.reshape(R)
    thr = jnp.full((L,), mask_threshold, dtype=jnp.float32)
    o0, o1 = _exchange_sc(xf, mf, thr)
    return (o0.reshape(2, 8192, D), o1.reshape(2, 8192, D))
